# baseline (speedup 1.0000x reference)
import functools

import jax
import jax.numpy as jnp
from jax import lax
from jax.experimental import pallas as pl
from jax.experimental.pallas import tpu as pltpu
from jax.experimental.pallas import tpu_sc as plsc

K = 2
BLK = 256
GTILE = 256
WREP = 16


def _make_router(NT, NP):
    def _router_tile(x_ref, wg_ref, bg_ref, g0_ref, p0_ref, p1_ref, te_ref,
                     counter, topi_s, rank_s):
        i = pl.program_id(0)
        E = wg_ref.shape[0]

        @pl.when(i == 0)
        def _():
            counter[...] = jnp.zeros_like(counter)

        @pl.when(i < NP)
        def _():
            xs = x_ref[...]
            scores = lax.dot_general(
                xs, wg_ref[...], (((1,), (1,)), ((), ())),
                preferred_element_type=jnp.float32,
            ) + bg_ref[...]

            col = lax.broadcasted_iota(jnp.int32, (GTILE, E), 1)
            m1 = jnp.max(scores, axis=1, keepdims=True)
            a1 = jnp.argmax(scores, axis=1).astype(jnp.int32)
            oh1 = col == a1[:, None]
            masked = jnp.where(oh1, -jnp.inf, scores)
            m2 = jnp.max(masked, axis=1, keepdims=True)
            a2 = jnp.argmax(masked, axis=1).astype(jnp.int32)
            oh2 = col == a2[:, None]

            t = jnp.exp(m2 - m1)
            g1 = 1.0 / (1.0 + t)

            occ = oh1.astype(jnp.float32) + oh2.astype(jnp.float32)
            ri = lax.broadcasted_iota(jnp.int32, (GTILE, GTILE), 0)
            ci = lax.broadcasted_iota(jnp.int32, (GTILE, GTILE), 1)
            ltri = (ci < ri).astype(jnp.float32)
            cumexc = lax.dot_general(
                ltri, occ, (((1,), (0,)), ((), ())),
                preferred_element_type=jnp.float32,
            ) + counter[...]
            r0 = jnp.sum(jnp.where(oh1, cumexc, 0.0), axis=1)
            r1 = jnp.sum(jnp.where(oh2, cumexc, 0.0), axis=1)

            counter[...] = counter[...] + jnp.sum(occ, axis=0, keepdims=True)
            base = i * GTILE
            topi_s[pl.ds(base, GTILE), :] = jnp.concatenate(
                [a1[:, None], a2[:, None]], axis=1)
            rank_s[pl.ds(base, GTILE), :] = jnp.concatenate(
                [r0[:, None], r1[:, None]], axis=1).astype(jnp.int32)
            g0_ref[...] = jnp.broadcast_to(g1, (GTILE, WREP))

        @pl.when(i >= NP)
        def _():
            c = counter[...]
            tiles = jnp.floor((c + (BLK - 1)) / BLK)
            ei = lax.broadcasted_iota(jnp.int32, (E, E), 0)
            ej = lax.broadcasted_iota(jnp.int32, (E, E), 1)
            cummat = (ei <= ej).astype(jnp.float32)
            tile_cum = lax.dot_general(
                tiles, cummat, (((1,), (0,)), ((), ())),
                preferred_element_type=jnp.float32,
            )
            pad_off = (tile_cum - tiles) * BLK

            base = (i - NP) * GTILE
            topi = topi_s[pl.ds(base, GTILE), :]
            rank = rank_s[pl.ds(base, GTILE), :]
            col = lax.broadcasted_iota(jnp.int32, (GTILE, E), 1)
            a1 = topi[:, 0][:, None]
            a2 = topi[:, 1][:, None]
            o1 = jnp.sum(jnp.where(col == a1, pad_off, 0.0), axis=1)
            o2 = jnp.sum(jnp.where(col == a2, pad_off, 0.0), axis=1)
            p0_ref[...] = o1.astype(jnp.int32)[:, None] + rank[:, 0][:, None]
            p1_ref[...] = o2.astype(jnp.int32)[:, None] + rank[:, 1][:, None]

            @pl.when(i == NP)
            def _():
                num_active = tile_cum[0, E - 1].astype(jnp.int32)
                ids = lax.broadcasted_iota(
                    jnp.int32, (NT, E), 0).astype(jnp.float32)
                cums = jnp.broadcast_to(tile_cum, (NT, E))
                tev = jnp.sum((cums <= ids).astype(jnp.float32), axis=1)
                tev = tev.astype(jnp.int32)
                eids = lax.broadcasted_iota(jnp.int32, (1, E), 1)
                last_e = jnp.max(
                    jnp.where(c > 0, eids, -1), axis=1)[0].astype(jnp.int32)
                tid = lax.broadcasted_iota(jnp.int32, (NT,), 0)
                tev = jnp.where(tid < num_active, tev, last_e)
                te_ref[...] = jnp.concatenate(
                    [tev, num_active[None]])[None, :]

    return _router_tile


def _ffn_tile(te_ref, xs_ref, W1_ref, b1_ref, W2_ref, b2_ref, out_ref):
    i = pl.program_id(0)
    nt = pl.num_programs(0)

    @pl.when(i < te_ref[nt])
    def _():
        xs = xs_ref[...].astype(jnp.float32)
        h = lax.dot_general(
            xs, W1_ref[0], (((1,), (1,)), ((), ())),
            preferred_element_type=jnp.float32,
        ) + b1_ref[0]
        h = 0.5 * h * (1.0 + lax.erf(h * 0.7071067811865476))
        out_ref[...] = lax.dot_general(
            h, W2_ref[0], (((1,), (1,)), ((), ())),
            preferred_element_type=jnp.float32,
        ) + b2_ref[0]


def kernel(x, Wg, bg, W1, b1, W2, b2):
    Bq, Sq, D = x.shape
    E, F, _ = W1.shape
    T = Bq * Sq
    A = T * K
    flat = x.reshape(T, D)
    NT = A // BLK + E
    PADDED = NT * BLK

    NP = T // GTILE
    g0w, p0, p1, te2 = pl.pallas_call(
        _make_router(NT, NP),
        grid=(2 * NP,),
        in_specs=[
            pl.BlockSpec((GTILE, D), lambda i: (jnp.minimum(i, NP - 1), 0)),
            pl.BlockSpec((E, D), lambda i: (0, 0)),
            pl.BlockSpec((1, E), lambda i: (0, 0)),
        ],
        out_specs=[
            pl.BlockSpec((GTILE, WREP), lambda i: (jnp.minimum(i, NP - 1), 0)),
            pl.BlockSpec((GTILE, 1), lambda i: (jnp.maximum(i - NP, 0), 0)),
            pl.BlockSpec((GTILE, 1), lambda i: (jnp.maximum(i - NP, 0), 0)),
            pl.BlockSpec((1, NT + 1), lambda i: (0, 0)),
        ],
        out_shape=[
            jax.ShapeDtypeStruct((T, WREP), jnp.float32),
            jax.ShapeDtypeStruct((T, 1), jnp.int32),
            jax.ShapeDtypeStruct((T, 1), jnp.int32),
            jax.ShapeDtypeStruct((1, NT + 1), jnp.int32),
        ],
        scratch_shapes=[
            pltpu.VMEM((1, E), jnp.float32),
            pltpu.VMEM((T, K), jnp.int32),
            pltpu.VMEM((T, K), jnp.int32),
        ],
    )(flat, Wg, bg.reshape(1, E))
    p0 = p0.reshape(T)
    p1 = p1.reshape(T)
    te = te2.reshape(NT + 1)

    info = plsc.get_sparse_core_info()
    NC, NS = info.num_cores, info.num_subcores
    NW = NC * NS
    mesh = plsc.VectorSubcoreMesh(core_axis_name="c", subcore_axis_name="s")
    tok_per_w = T // NW

    SUBT = 64
    NSUB = tok_per_w // SUBT

    @functools.partial(
        pl.kernel,
        out_type=jax.ShapeDtypeStruct((PADDED, D // 2), jnp.int32),
        mesh=mesh,
        scratch_types=[
            pltpu.VMEM((NSUB, SUBT), jnp.int32),
            pltpu.VMEM((NSUB, SUBT), jnp.int32),
            pltpu.VMEM((2, SUBT, D // 2), jnp.int32),
            pltpu.SemaphoreType.DMA,
            pltpu.SemaphoreType.DMA,
        ],
    )
    def _dispatch(flat_hbm, p0_hbm, p1_hbm, xs_hbm,
                  p0_v, p1_v, rows_v, sem_in, sem_out):
        wid = lax.axis_index("s") * NC + lax.axis_index("c")
        tbase = wid * tok_per_w
        for j in range(NSUB):
            pltpu.sync_copy(p0_hbm.at[pl.ds(tbase + j * SUBT, SUBT)], p0_v.at[j])
            pltpu.sync_copy(p1_hbm.at[pl.ds(tbase + j * SUBT, SUBT)], p1_v.at[j])

        def issue_in(j):
            b = j % 2
            return pltpu.async_copy(
                flat_hbm.at[pl.ds(tbase + j * SUBT, SUBT)], rows_v.at[b],
                sem_in)

        def issue_out(j):
            b = j % 2
            return (
                pltpu.async_copy(rows_v.at[b], xs_hbm.at[p0_v.at[j]], sem_out),
                pltpu.async_copy(rows_v.at[b], xs_hbm.at[p1_v.at[j]], sem_out),
            )

        pend_in = issue_in(0)
        pend_out = None
        for j in range(NSUB):
            pend_in.wait()
            if j + 1 < NSUB:
                if pend_out is not None:
                    for d in pend_out:
                        d.wait()
                    pend_out = None
                pend_in = issue_in(j + 1)
            if pend_out is not None:
                for d in pend_out:
                    d.wait()
            pend_out = issue_out(j)
        for d in pend_out:
            d.wait()

    flat_packed = lax.bitcast_convert_type(
        flat.astype(jnp.bfloat16).reshape(T, D // 2, 2), jnp.int32)
    xs_i32 = _dispatch(flat_packed, p0, p1)
    xs = lax.bitcast_convert_type(xs_i32, jnp.bfloat16).reshape(PADDED, D)

    def _row_ix(i, te):
        return (jnp.minimum(i, te[NT] - 1), 0)

    grid_spec = pltpu.PrefetchScalarGridSpec(
        num_scalar_prefetch=1,
        grid=(NT,),
        in_specs=[
            pl.BlockSpec((BLK, D), _row_ix),
            pl.BlockSpec((1, F, D), lambda i, te: (te[i], 0, 0)),
            pl.BlockSpec((1, 1, F), lambda i, te: (te[i], 0, 0)),
            pl.BlockSpec((1, D, F), lambda i, te: (te[i], 0, 0)),
            pl.BlockSpec((1, 1, D), lambda i, te: (te[i], 0, 0)),
        ],
        out_specs=pl.BlockSpec((BLK, D), _row_ix),
    )
    ys = pl.pallas_call(
        _ffn_tile,
        grid_spec=grid_spec,
        out_shape=jax.ShapeDtypeStruct((PADDED, D), jnp.float32),
    )(te, xs, W1, b1.reshape(E, 1, F), W2, b2.reshape(E, 1, D))

    SUBC = 32
    NSUBC = tok_per_w // SUBC

    @functools.partial(
        pl.kernel,
        out_type=jax.ShapeDtypeStruct((T, D), jnp.float32),
        mesh=mesh,
        scratch_types=[
            pltpu.VMEM((NSUBC, SUBC), jnp.int32),
            pltpu.VMEM((NSUBC, SUBC), jnp.int32),
            pltpu.VMEM((2, SUBC, D), jnp.float32),
            pltpu.VMEM((2, SUBC, D), jnp.float32),
            pltpu.VMEM((2, SUBC, WREP), jnp.float32),
            pltpu.SemaphoreType.DMA,
            pltpu.SemaphoreType.DMA,
        ],
    )
    def _combine(ys_hbm, p0_hbm, p1_hbm, g0_hbm, out_hbm,
                 p0_v, p1_v, rows0_v, rows1_v, g0_v, sem_in, sem_out):
        wid = lax.axis_index("s") * NC + lax.axis_index("c")
        tbase = wid * tok_per_w
        L = 16
        for j in range(NSUBC):
            pltpu.sync_copy(p0_hbm.at[pl.ds(tbase + j * SUBC, SUBC)], p0_v.at[j])
            pltpu.sync_copy(p1_hbm.at[pl.ds(tbase + j * SUBC, SUBC)], p1_v.at[j])

        def issue_in(j):
            b = j % 2
            return (
                pltpu.async_copy(ys_hbm.at[p0_v.at[j]], rows0_v.at[b], sem_in),
                pltpu.async_copy(ys_hbm.at[p1_v.at[j]], rows1_v.at[b], sem_in),
                pltpu.async_copy(
                    g0_hbm.at[pl.ds(tbase + j * SUBC, SUBC)], g0_v.at[b], sem_in),
            )

        pend_in = issue_in(0)
        pend_out = None
        for j in range(NSUBC):
            b = j % 2
            for d in pend_in:
                d.wait()
            if j + 1 < NSUBC:
                if pend_out is not None:
                    pend_out.wait()
                    pend_out = None
                pend_in = issue_in(j + 1)

            def body(r, _):
                b0 = g0_v[b, r, pl.ds(0, L)]
                b1 = 1.0 - b0
                for c in range(D // L):
                    r0 = rows0_v[b, r, pl.ds(c * L, L)]
                    r1 = rows1_v[b, r, pl.ds(c * L, L)]
                    rows0_v[b, r, pl.ds(c * L, L)] = r0 * b0 + r1 * b1
                return 0

            lax.fori_loop(0, SUBC, body, 0, unroll=2)
            if pend_out is not None:
                pend_out.wait()
            pend_out = pltpu.async_copy(
                rows0_v.at[b], out_hbm.at[pl.ds(tbase + j * SUBC, SUBC)],
                sem_out)
        pend_out.wait()

    out = _combine(ys, p0, p1, g0w)
    return out.reshape(Bq, Sq, D)


# speedup vs baseline: 1.8801x; 1.8801x over previous
import functools

import jax
import jax.numpy as jnp
from jax import lax
from jax.experimental import pallas as pl
from jax.experimental.pallas import tpu as pltpu
from jax.experimental.pallas import tpu_sc as plsc

K = 2
BLK = 256
GTILE = 256
WREP = 16


def _make_router(NT, NP):
    def _router_tile(x_ref, wg_ref, bg_ref, g0_ref, p0_ref, p1_ref, te_ref,
                     counter, topi_s, rank_s):
        i = pl.program_id(0)
        E = wg_ref.shape[0]

        @pl.when(i == 0)
        def _():
            counter[...] = jnp.zeros_like(counter)

        @pl.when(i < NP)
        def _():
            xs = x_ref[...]
            scores = lax.dot_general(
                xs, wg_ref[...], (((1,), (1,)), ((), ())),
                preferred_element_type=jnp.float32,
            ) + bg_ref[...]

            col = lax.broadcasted_iota(jnp.int32, (GTILE, E), 1)
            m1 = jnp.max(scores, axis=1, keepdims=True)
            a1 = jnp.argmax(scores, axis=1).astype(jnp.int32)
            oh1 = col == a1[:, None]
            masked = jnp.where(oh1, -jnp.inf, scores)
            m2 = jnp.max(masked, axis=1, keepdims=True)
            a2 = jnp.argmax(masked, axis=1).astype(jnp.int32)
            oh2 = col == a2[:, None]

            t = jnp.exp(m2 - m1)
            g1 = 1.0 / (1.0 + t)

            occ = oh1.astype(jnp.float32) + oh2.astype(jnp.float32)
            ri = lax.broadcasted_iota(jnp.int32, (GTILE, GTILE), 0)
            ci = lax.broadcasted_iota(jnp.int32, (GTILE, GTILE), 1)
            ltri = (ci < ri).astype(jnp.float32)
            cumexc = lax.dot_general(
                ltri, occ, (((1,), (0,)), ((), ())),
                preferred_element_type=jnp.float32,
            ) + counter[...]
            r0 = jnp.sum(jnp.where(oh1, cumexc, 0.0), axis=1)
            r1 = jnp.sum(jnp.where(oh2, cumexc, 0.0), axis=1)

            counter[...] = counter[...] + jnp.sum(occ, axis=0, keepdims=True)
            base = i * GTILE
            topi_s[pl.ds(base, GTILE), :] = jnp.concatenate(
                [a1[:, None], a2[:, None]], axis=1)
            rank_s[pl.ds(base, GTILE), :] = jnp.concatenate(
                [r0[:, None], r1[:, None]], axis=1).astype(jnp.int32)
            g0_ref[...] = jnp.broadcast_to(g1, (GTILE, WREP))

        @pl.when(i >= NP)
        def _():
            c = counter[...]
            tiles = jnp.floor((c + (BLK - 1)) / BLK)
            ei = lax.broadcasted_iota(jnp.int32, (E, E), 0)
            ej = lax.broadcasted_iota(jnp.int32, (E, E), 1)
            cummat = (ei <= ej).astype(jnp.float32)
            tile_cum = lax.dot_general(
                tiles, cummat, (((1,), (0,)), ((), ())),
                preferred_element_type=jnp.float32,
            )
            pad_off = (tile_cum - tiles) * BLK

            base = (i - NP) * GTILE
            topi = topi_s[pl.ds(base, GTILE), :]
            rank = rank_s[pl.ds(base, GTILE), :]
            col = lax.broadcasted_iota(jnp.int32, (GTILE, E), 1)
            a1 = topi[:, 0][:, None]
            a2 = topi[:, 1][:, None]
            o1 = jnp.sum(jnp.where(col == a1, pad_off, 0.0), axis=1)
            o2 = jnp.sum(jnp.where(col == a2, pad_off, 0.0), axis=1)
            p0_ref[...] = o1.astype(jnp.int32)[:, None] + rank[:, 0][:, None]
            p1_ref[...] = o2.astype(jnp.int32)[:, None] + rank[:, 1][:, None]

            @pl.when(i == NP)
            def _():
                num_active = tile_cum[0, E - 1].astype(jnp.int32)
                ids = lax.broadcasted_iota(
                    jnp.int32, (NT, E), 0).astype(jnp.float32)
                cums = jnp.broadcast_to(tile_cum, (NT, E))
                tev = jnp.sum((cums <= ids).astype(jnp.float32), axis=1)
                tev = tev.astype(jnp.int32)
                eids = lax.broadcasted_iota(jnp.int32, (1, E), 1)
                last_e = jnp.max(
                    jnp.where(c > 0, eids, -1), axis=1)[0].astype(jnp.int32)
                tid = lax.broadcasted_iota(jnp.int32, (NT,), 0)
                tev = jnp.where(tid < num_active, tev, last_e)
                te_ref[...] = jnp.concatenate(
                    [tev, num_active[None]])[None, :]

    return _router_tile


def _ffn_tile(te_ref, xs_ref, W1_ref, b1_ref, W2_ref, b2_ref, out_ref):
    i = pl.program_id(0)
    nt = pl.num_programs(0)

    @pl.when(i < te_ref[nt])
    def _():
        xs = xs_ref[...]
        h = lax.dot_general(
            xs, W1_ref[0], (((1,), (1,)), ((), ())),
            preferred_element_type=jnp.float32,
        ) + b1_ref[0]
        h = 0.5 * h * (1.0 + lax.erf(h * 0.7071067811865476))
        out_ref[...] = lax.dot_general(
            h, W2_ref[0], (((1,), (1,)), ((), ())),
            preferred_element_type=jnp.float32,
        ) + b2_ref[0]


def kernel(x, Wg, bg, W1, b1, W2, b2):
    Bq, Sq, D = x.shape
    E, F, _ = W1.shape
    T = Bq * Sq
    A = T * K
    flat = x.reshape(T, D)
    NT = A // BLK + E
    PADDED = NT * BLK

    NP = T // GTILE
    g0w, p0, p1, te2 = pl.pallas_call(
        _make_router(NT, NP),
        grid=(2 * NP,),
        in_specs=[
            pl.BlockSpec((GTILE, D), lambda i: (jnp.minimum(i, NP - 1), 0)),
            pl.BlockSpec((E, D), lambda i: (0, 0)),
            pl.BlockSpec((1, E), lambda i: (0, 0)),
        ],
        out_specs=[
            pl.BlockSpec((GTILE, WREP), lambda i: (jnp.minimum(i, NP - 1), 0)),
            pl.BlockSpec((GTILE, 1), lambda i: (jnp.maximum(i - NP, 0), 0)),
            pl.BlockSpec((GTILE, 1), lambda i: (jnp.maximum(i - NP, 0), 0)),
            pl.BlockSpec((1, NT + 1), lambda i: (0, 0)),
        ],
        out_shape=[
            jax.ShapeDtypeStruct((T, WREP), jnp.float32),
            jax.ShapeDtypeStruct((T, 1), jnp.int32),
            jax.ShapeDtypeStruct((T, 1), jnp.int32),
            jax.ShapeDtypeStruct((1, NT + 1), jnp.int32),
        ],
        scratch_shapes=[
            pltpu.VMEM((1, E), jnp.float32),
            pltpu.VMEM((T, K), jnp.int32),
            pltpu.VMEM((T, K), jnp.int32),
        ],
    )(flat, Wg, bg.reshape(1, E))
    p0 = p0.reshape(T)
    p1 = p1.reshape(T)
    te = te2.reshape(NT + 1)

    info = plsc.get_sparse_core_info()
    NC, NS = info.num_cores, info.num_subcores
    NW = NC * NS
    mesh = plsc.VectorSubcoreMesh(core_axis_name="c", subcore_axis_name="s")
    tok_per_w = T // NW

    SUBT = 64
    NSUB = tok_per_w // SUBT

    @functools.partial(
        pl.kernel,
        out_type=jax.ShapeDtypeStruct((PADDED, D), jnp.float32),
        mesh=mesh,
        scratch_types=[
            pltpu.VMEM((NSUB, SUBT), jnp.int32),
            pltpu.VMEM((NSUB, SUBT), jnp.int32),
            pltpu.VMEM((2, SUBT, D), jnp.float32),
            pltpu.SemaphoreType.DMA,
            pltpu.SemaphoreType.DMA,
        ],
    )
    def _dispatch(flat_hbm, p0_hbm, p1_hbm, xs_hbm,
                  p0_v, p1_v, rows_v, sem_in, sem_out):
        wid = lax.axis_index("s") * NC + lax.axis_index("c")
        tbase = wid * tok_per_w
        for j in range(NSUB):
            pltpu.sync_copy(p0_hbm.at[pl.ds(tbase + j * SUBT, SUBT)], p0_v.at[j])
            pltpu.sync_copy(p1_hbm.at[pl.ds(tbase + j * SUBT, SUBT)], p1_v.at[j])

        def issue_in(j):
            b = j % 2
            return pltpu.async_copy(
                flat_hbm.at[pl.ds(tbase + j * SUBT, SUBT)], rows_v.at[b],
                sem_in)

        def issue_out(j):
            b = j % 2
            return (
                pltpu.async_copy(rows_v.at[b], xs_hbm.at[p0_v.at[j]], sem_out),
                pltpu.async_copy(rows_v.at[b], xs_hbm.at[p1_v.at[j]], sem_out),
            )

        pend_in = issue_in(0)
        pend_out = None
        for j in range(NSUB):
            pend_in.wait()
            if j + 1 < NSUB:
                if pend_out is not None:
                    for d in pend_out:
                        d.wait()
                    pend_out = None
                pend_in = issue_in(j + 1)
            if pend_out is not None:
                for d in pend_out:
                    d.wait()
            pend_out = issue_out(j)
        for d in pend_out:
            d.wait()

    xs = _dispatch(flat, p0, p1)

    def _row_ix(i, te):
        return (jnp.minimum(i, te[NT] - 1), 0)

    grid_spec = pltpu.PrefetchScalarGridSpec(
        num_scalar_prefetch=1,
        grid=(NT,),
        in_specs=[
            pl.BlockSpec((BLK, D), _row_ix),
            pl.BlockSpec((1, F, D), lambda i, te: (te[i], 0, 0)),
            pl.BlockSpec((1, 1, F), lambda i, te: (te[i], 0, 0)),
            pl.BlockSpec((1, D, F), lambda i, te: (te[i], 0, 0)),
            pl.BlockSpec((1, 1, D), lambda i, te: (te[i], 0, 0)),
        ],
        out_specs=pl.BlockSpec((BLK, D), _row_ix),
    )
    ys = pl.pallas_call(
        _ffn_tile,
        grid_spec=grid_spec,
        out_shape=jax.ShapeDtypeStruct((PADDED, D), jnp.float32),
    )(te, xs, W1, b1.reshape(E, 1, F), W2, b2.reshape(E, 1, D))

    SUBC = 32
    NSUBC = tok_per_w // SUBC

    @functools.partial(
        pl.kernel,
        out_type=jax.ShapeDtypeStruct((T, D), jnp.float32),
        mesh=mesh,
        scratch_types=[
            pltpu.VMEM((NSUBC, SUBC), jnp.int32),
            pltpu.VMEM((NSUBC, SUBC), jnp.int32),
            pltpu.VMEM((2, SUBC, D), jnp.float32),
            pltpu.VMEM((2, SUBC, D), jnp.float32),
            pltpu.VMEM((2, SUBC, WREP), jnp.float32),
            pltpu.SemaphoreType.DMA,
            pltpu.SemaphoreType.DMA,
        ],
    )
    def _combine(ys_hbm, p0_hbm, p1_hbm, g0_hbm, out_hbm,
                 p0_v, p1_v, rows0_v, rows1_v, g0_v, sem_in, sem_out):
        wid = lax.axis_index("s") * NC + lax.axis_index("c")
        tbase = wid * tok_per_w
        L = 16
        for j in range(NSUBC):
            pltpu.sync_copy(p0_hbm.at[pl.ds(tbase + j * SUBC, SUBC)], p0_v.at[j])
            pltpu.sync_copy(p1_hbm.at[pl.ds(tbase + j * SUBC, SUBC)], p1_v.at[j])

        def issue_in(j):
            b = j % 2
            return (
                pltpu.async_copy(ys_hbm.at[p0_v.at[j]], rows0_v.at[b], sem_in),
                pltpu.async_copy(ys_hbm.at[p1_v.at[j]], rows1_v.at[b], sem_in),
                pltpu.async_copy(
                    g0_hbm.at[pl.ds(tbase + j * SUBC, SUBC)], g0_v.at[b], sem_in),
            )

        pend_in = issue_in(0)
        pend_out = None
        for j in range(NSUBC):
            b = j % 2
            for d in pend_in:
                d.wait()
            if j + 1 < NSUBC:
                if pend_out is not None:
                    pend_out.wait()
                    pend_out = None
                pend_in = issue_in(j + 1)

            def body(r, _):
                b0 = g0_v[b, r, pl.ds(0, L)]
                b1 = 1.0 - b0
                for c in range(D // L):
                    r0 = rows0_v[b, r, pl.ds(c * L, L)]
                    r1 = rows1_v[b, r, pl.ds(c * L, L)]
                    rows0_v[b, r, pl.ds(c * L, L)] = r0 * b0 + r1 * b1
                return 0

            lax.fori_loop(0, SUBC, body, 0, unroll=2)
            if pend_out is not None:
                pend_out.wait()
            pend_out = pltpu.async_copy(
                rows0_v.at[b], out_hbm.at[pl.ds(tbase + j * SUBC, SUBC)],
                sem_out)
        pend_out.wait()

    out = _combine(ys, p0, p1, g0w)
    return out.reshape(Bq, Sq, D)
